# baseline (speedup 1.0000x reference)
import functools

import jax
import jax.numpy as jnp
from jax import lax
from jax.experimental import pallas as pl
from jax.experimental.pallas import tpu as pltpu
from jax.experimental.pallas import tpu_sc as plsc

_B = 16384
_D = 128
_F = 13
_W = _F * _D


def _build_plan():
    plan = []
    off = [0, 0]
    for f in range(2 * _F):
        in_idx, c = divmod(f, _F)
        o = f % 2
        plan.append((o, off[o], in_idx, c))
        off[o] += 1
    return tuple(plan)


_PLAN = _build_plan()

_NC = 2
_NS = 16
_NW = _NC * _NS
_RPW = _B // _NW


@functools.partial(
    pl.kernel,
    mesh=plsc.VectorSubcoreMesh(core_axis_name="c", subcore_axis_name="s"),
    out_type=(
        jax.ShapeDtypeStruct((_B, _W), jnp.float32),
        jax.ShapeDtypeStruct((_B, _W), jnp.float32),
    ),
    scratch_types=[
        pltpu.SemaphoreType.DMA,
    ],
)
def _permute_sc(v0_hbm, v1_hbm, o0_hbm, o1_hbm, sem):
    wid = lax.axis_index("s") * _NC + lax.axis_index("c")
    base = wid * _RPW
    srcs = (v0_hbm, v1_hbm)
    outs = (o0_hbm, o1_hbm)
    handles = []
    for (o, j, s, c) in _PLAN:
        handles.append(pltpu.async_copy(
            srcs[s].at[pl.ds(base, _RPW), pl.ds(c * _D, _D)],
            outs[o].at[pl.ds(base, _RPW), pl.ds(j * _D, _D)],
            sem))
    for h in handles:
        h.wait()


def kernel(v0, v1):
    return _permute_sc(v0, v1)


# speedup vs baseline: 35.7148x; 35.7148x over previous
import functools

import jax
import jax.numpy as jnp
from jax import lax
from jax.experimental import pallas as pl
from jax.experimental.pallas import tpu as pltpu
from jax.experimental.pallas import tpu_sc as plsc

_B = 16384
_D = 128
_F = 13
_W = _F * _D


def _build_plan():
    plan = []
    off = [0, 0]
    for f in range(2 * _F):
        in_idx, c = divmod(f, _F)
        o = f % 2
        plan.append((o, off[o], in_idx, c))
        off[o] += 1
    return tuple(plan)


_PLAN = _build_plan()

_NC = 2
_NS = 16
_NW = _NC * _NS
_RPW = _B // _NW
_R = 16
_CHUNKS = _RPW // _R


@functools.partial(
    pl.kernel,
    mesh=plsc.VectorSubcoreMesh(core_axis_name="c", subcore_axis_name="s"),
    out_type=(
        jax.ShapeDtypeStruct((_B, _W), jnp.float32),
        jax.ShapeDtypeStruct((_B, _W), jnp.float32),
    ),
    scratch_types=[
        pltpu.VMEM((_R, _W), jnp.float32),
        pltpu.VMEM((_R, _W), jnp.float32),
        pltpu.VMEM((_R, _W), jnp.float32),
        pltpu.VMEM((_R, _W), jnp.float32),
        pltpu.SemaphoreType.DMA,
        pltpu.SemaphoreType.DMA,
        pltpu.SemaphoreType.DMA,
    ],
)
def _permute_sc(v0_hbm, v1_hbm, o0_hbm, o1_hbm,
                in0_a, in1_a, in0_b, in1_b, sem_r, sem_wa, sem_wb):
    wid = lax.axis_index("s") * _NC + lax.axis_index("c")
    base = wid * _RPW
    srcs = (v0_hbm, v1_hbm)
    outs = (o0_hbm, o1_hbm)
    bufs = ((in0_a, in1_a), (in0_b, in1_b))
    sem_w = (sem_wa, sem_wb)

    def fire_reads(r0, b):
        return [
            pltpu.async_copy(srcs[s].at[pl.ds(r0, _R), :], bufs[b][s], sem_r)
            for s in range(2)
        ]

    def fire_writes(r0, b):
        for (o, j, s, c) in _PLAN:
            pltpu.async_copy(
                bufs[b][s].at[:, pl.ds(c * _D, _D)],
                outs[o].at[pl.ds(r0, _R), pl.ds(j * _D, _D)],
                sem_w[b])

    def drain_writes(b):
        for (o, j, s, c) in _PLAN:
            pltpu.make_async_copy(
                bufs[b][s].at[:, pl.ds(c * _D, _D)],
                outs[o].at[pl.ds(base, _R), pl.ds(j * _D, _D)],
                sem_w[b]).wait()

    def chunk(i, b):
        r0 = base + i * _R
        reads = fire_reads(r0, b)
        for h in reads:
            h.wait()
        fire_writes(r0, b)

    chunk(0, 0)
    chunk(1, 1)

    def body(g, carry):
        r0a = base + (2 * g) * _R
        r0b = base + (2 * g + 1) * _R
        for b, r0 in ((0, r0a), (1, r0b)):
            drain_writes(b)
            reads = fire_reads(r0, b)
            for h in reads:
                h.wait()
            fire_writes(r0, b)
        return carry

    lax.fori_loop(1, _CHUNKS // 2, body, 0)

    drain_writes(0)
    drain_writes(1)


def kernel(v0, v1):
    return _permute_sc(v0, v1)


# speedup vs baseline: 36.0713x; 1.0100x over previous
import functools

import jax
import jax.numpy as jnp
from jax import lax
from jax.experimental import pallas as pl
from jax.experimental.pallas import tpu as pltpu
from jax.experimental.pallas import tpu_sc as plsc

_B = 16384
_D = 128
_F = 13
_W = _F * _D


def _build_plan():
    plan = []
    off = [0, 0]
    for f in range(2 * _F):
        in_idx, c = divmod(f, _F)
        o = f % 2
        plan.append((o, off[o], in_idx, c))
        off[o] += 1
    return tuple(plan)


_PLAN = _build_plan()

_NC = 2
_NS = 16
_NW = _NC * _NS
_RPW = _B // _NW
_R = 16
_CHUNKS = _RPW // _R


@functools.partial(
    pl.kernel,
    mesh=plsc.VectorSubcoreMesh(core_axis_name="c", subcore_axis_name="s"),
    out_type=(
        jax.ShapeDtypeStruct((_B, _W), jnp.float32),
        jax.ShapeDtypeStruct((_B, _W), jnp.float32),
    ),
    scratch_types=[
        pltpu.VMEM((_R, _W), jnp.float32),
        pltpu.VMEM((_R, _W), jnp.float32),
        pltpu.VMEM((_R, _W), jnp.float32),
        pltpu.VMEM((_R, _W), jnp.float32),
        pltpu.SemaphoreType.DMA,
        pltpu.SemaphoreType.DMA,
        pltpu.SemaphoreType.DMA,
    ],
)
def _permute_sc(v0_hbm, v1_hbm, o0_hbm, o1_hbm,
                in0_a, in1_a, in0_b, in1_b, sem_r, sem_wa, sem_wb):
    wid = lax.axis_index("s") * _NC + lax.axis_index("c")
    base = wid * _RPW
    srcs = (v0_hbm, v1_hbm)
    outs = (o0_hbm, o1_hbm)
    bufs = ((in0_a, in1_a), (in0_b, in1_b))
    sem_w = (sem_wa, sem_wb)

    def fire_reads(r0, b):
        return [
            pltpu.async_copy(srcs[s].at[pl.ds(r0, _R), :], bufs[b][s], sem_r)
            for s in range(2)
        ]

    def fire_writes(r0, b):
        for (o, j, s, c) in _PLAN:
            pltpu.async_copy(
                bufs[b][s].at[:, pl.ds(c * _D, _D)],
                outs[o].at[pl.ds(r0, _R), pl.ds(j * _D, _D)],
                sem_w[b])

    def drain_writes(b):
        for s in range(2):
            pltpu.make_async_copy(
                bufs[b][s],
                outs[s].at[pl.ds(base, _R), :],
                sem_w[b]).wait()

    def chunk(i, b):
        r0 = base + i * _R
        reads = fire_reads(r0, b)
        for h in reads:
            h.wait()
        fire_writes(r0, b)

    chunk(0, 0)
    chunk(1, 1)

    def body(g, carry):
        r0a = base + (2 * g) * _R
        r0b = base + (2 * g + 1) * _R
        for b, r0 in ((0, r0a), (1, r0b)):
            drain_writes(b)
            reads = fire_reads(r0, b)
            for h in reads:
                h.wait()
            fire_writes(r0, b)
        return carry

    lax.fori_loop(1, _CHUNKS // 2, body, 0)

    drain_writes(0)
    drain_writes(1)


def kernel(v0, v1):
    return _permute_sc(v0, v1)
